# Initial kernel scaffold; baseline (speedup 1.0000x reference)
import jax
import jax.numpy as jnp
from jax.experimental import pallas as pl


def kernel(item_raw_features, u_embedding, i_embedding, W, b):
    raise NotImplementedError("write your pallas kernel here")


# speedup vs baseline: 1.1280x; 1.1280x over previous
import jax
import jax.numpy as jnp
from jax.experimental import pallas as pl
from jax.experimental.pallas import tpu as pltpu

N_ROWS = 100000
BM = 1000
EMB = 128
FEAT = 1024


def _body(raw_ref, u_ref, i_ref, w_ref, b_ref, uo_ref, io_ref):
    uo_ref[...] = u_ref[...]
    io_ref[:, :EMB] = i_ref[...]
    proj = jnp.dot(raw_ref[...], w_ref[...], preferred_element_type=jnp.float32)
    io_ref[:, EMB:] = proj + b_ref[...]


def kernel(item_raw_features, u_embedding, i_embedding, W, b):
    b2 = b.reshape(1, EMB)
    grid = (N_ROWS // BM,)
    user_e, item_e = pl.pallas_call(
        _body,
        grid=grid,
        in_specs=[
            pl.BlockSpec((BM, FEAT), lambda i: (i, 0)),
            pl.BlockSpec((BM, 2 * EMB), lambda i: (i, 0)),
            pl.BlockSpec((BM, EMB), lambda i: (i, 0)),
            pl.BlockSpec((FEAT, EMB), lambda i: (0, 0)),
            pl.BlockSpec((1, EMB), lambda i: (0, 0)),
        ],
        out_specs=[
            pl.BlockSpec((BM, 2 * EMB), lambda i: (i, 0)),
            pl.BlockSpec((BM, 2 * EMB), lambda i: (i, 0)),
        ],
        out_shape=[
            jax.ShapeDtypeStruct((N_ROWS, 2 * EMB), jnp.float32),
            jax.ShapeDtypeStruct((N_ROWS, 2 * EMB), jnp.float32),
        ],
        compiler_params=pltpu.CompilerParams(
            dimension_semantics=("arbitrary",),
        ),
    )(item_raw_features, u_embedding, i_embedding, W, b2)
    return (user_e, item_e)


# speedup vs baseline: 1.1637x; 1.0317x over previous
import jax
import jax.numpy as jnp
from jax.experimental import pallas as pl
from jax.experimental.pallas import tpu as pltpu

N_ROWS = 100000
BM = 2000
EMB = 128
FEAT = 1024


def _body(raw_ref, u_ref, i_ref, w_ref, b_ref, uo_ref, io_ref):
    uo_ref[...] = u_ref[...]
    io_ref[:, :EMB] = i_ref[...]
    proj = jnp.dot(raw_ref[...], w_ref[...], preferred_element_type=jnp.float32)
    io_ref[:, EMB:] = proj + b_ref[...]


def kernel(item_raw_features, u_embedding, i_embedding, W, b):
    b2 = b.reshape(1, EMB)
    grid = (N_ROWS // BM,)
    user_e, item_e = pl.pallas_call(
        _body,
        grid=grid,
        in_specs=[
            pl.BlockSpec((BM, FEAT), lambda i: (i, 0)),
            pl.BlockSpec((BM, 2 * EMB), lambda i: (i, 0)),
            pl.BlockSpec((BM, EMB), lambda i: (i, 0)),
            pl.BlockSpec((FEAT, EMB), lambda i: (0, 0)),
            pl.BlockSpec((1, EMB), lambda i: (0, 0)),
        ],
        out_specs=[
            pl.BlockSpec((BM, 2 * EMB), lambda i: (i, 0)),
            pl.BlockSpec((BM, 2 * EMB), lambda i: (i, 0)),
        ],
        out_shape=[
            jax.ShapeDtypeStruct((N_ROWS, 2 * EMB), jnp.float32),
            jax.ShapeDtypeStruct((N_ROWS, 2 * EMB), jnp.float32),
        ],
        compiler_params=pltpu.CompilerParams(
            dimension_semantics=("arbitrary",),
        ),
    )(item_raw_features, u_embedding, i_embedding, W, b2)
    return (user_e, item_e)
